# baseline (speedup 1.0000x reference)
import functools

import jax
import jax.numpy as jnp
from jax import lax
from jax.experimental import pallas as pl
from jax.experimental.pallas import tpu as pltpu
from jax.experimental.pallas import tpu_sc as plsc

NUM_ROWS = 1000000
DIM = 16
NFEAT = 26
NB = 16384

NUM_CORES = 2
NUM_SUBCORES = 16
NUM_WORKERS = NUM_CORES * NUM_SUBCORES
B_PER_W = NB // NUM_WORKERS
BCHUNK = 128
NCHUNK = B_PER_W // BCHUNK

_mesh = plsc.VectorSubcoreMesh(
    core_axis_name="c", subcore_axis_name="s",
    num_cores=NUM_CORES, num_subcores=NUM_SUBCORES)


@functools.partial(
    pl.kernel,
    out_type=jax.ShapeDtypeStruct((NFEAT, DIM, NB), jnp.float32),
    mesh=_mesh,
    scratch_types=(
        pltpu.VMEM((NFEAT, BCHUNK), jnp.int32),
        pltpu.VMEM((NFEAT, DIM, BCHUNK), jnp.float32),
        pltpu.SemaphoreType.DMA,
    ),
    compiler_params=pltpu.CompilerParams(use_tc_tiling_on_sc=False),
)
def _gather_kernel(xt_hbm, tt_hbm, out_hbm, idx_v, out_buf, gsem):
    wid = lax.axis_index("s") * NUM_CORES + lax.axis_index("c")
    base = wid * B_PER_W

    def chunk_body(c, carry):
        b0 = base + c * BCHUNK
        pltpu.sync_copy(xt_hbm.at[:, pl.ds(b0, BCHUNK)], idx_v)

        def fire(jd, carry2):
            j = jd // DIM
            d = jd % DIM
            pltpu.async_copy(
                tt_hbm.at[d].at[idx_v.at[j]], out_buf.at[j, d], gsem)
            return carry2

        lax.fori_loop(0, NFEAT * DIM, fire, 0)
        pltpu.make_async_copy(
            out_hbm.at[:, :, pl.ds(0, BCHUNK)], out_buf, gsem).wait()
        pltpu.sync_copy(out_buf, out_hbm.at[:, :, pl.ds(b0, BCHUNK)])
        return carry

    lax.fori_loop(0, NCHUNK, chunk_body, 0)


def kernel(x, table):
    out_t = _gather_kernel(x.T, table.T)
    return jnp.transpose(out_t, (2, 0, 1))


# speedup vs baseline: 2.1892x; 2.1892x over previous
import functools

import jax
import jax.numpy as jnp
from jax import lax
from jax.experimental import pallas as pl
from jax.experimental.pallas import tpu as pltpu
from jax.experimental.pallas import tpu_sc as plsc

NUM_ROWS = 1000000
DIM = 16
NFEAT = 26
NB = 16384

NUM_CORES = 2
NUM_SUBCORES = 16
NUM_WORKERS = NUM_CORES * NUM_SUBCORES

_mesh = plsc.VectorSubcoreMesh(
    core_axis_name="c", subcore_axis_name="s",
    num_cores=NUM_CORES, num_subcores=NUM_SUBCORES)

VCHUNK = 512
N_FULL = NUM_ROWS // VCHUNK
TAIL = NUM_ROWS - N_FULL * VCHUNK
CHUNKS_PER_W = (N_FULL + 1 + NUM_WORKERS - 1) // NUM_WORKERS


@functools.partial(
    pl.kernel,
    out_type=jax.ShapeDtypeStruct((NUM_ROWS * DIM,), jnp.float32),
    mesh=_mesh,
    scratch_types=(
        pltpu.VMEM((DIM, VCHUNK), jnp.float32),
        pltpu.VMEM((VCHUNK * DIM,), jnp.float32),
    ),
    compiler_params=pltpu.CompilerParams(needs_layout_passes=False),
)
def _relayout_kernel(tt_hbm, tail_hbm, out_hbm, in_buf, row_buf):
    wid = lax.axis_index("s") * NUM_CORES + lax.axis_index("c")
    lane = lax.iota(jnp.int32, 16)

    def do_chunk(v0):
        pltpu.sync_copy(tt_hbm.at[:, pl.ds(v0, VCHUNK)], in_buf)

        def tr(v, carry):
            row = plsc.load_gather(in_buf, [lane, jnp.full((16,), v, jnp.int32)])
            row_buf[pl.ds(v * DIM, DIM)] = row
            return carry

        lax.fori_loop(0, VCHUNK, tr, 0)
        pltpu.sync_copy(row_buf, out_hbm.at[pl.ds(v0 * DIM, VCHUNK * DIM)])

    def chunk_body(i, carry):
        ci = wid + NUM_WORKERS * i

        @pl.when(ci < N_FULL)
        def _():
            do_chunk(ci * VCHUNK)

        @pl.when(ci == N_FULL)
        def _():
            pltpu.sync_copy(tail_hbm, row_buf.at[pl.ds(0, TAIL * DIM)])
            pltpu.sync_copy(row_buf.at[pl.ds(0, TAIL * DIM)],
                            out_hbm.at[pl.ds(N_FULL * VCHUNK * DIM, TAIL * DIM)])

        return carry

    lax.fori_loop(0, CHUNKS_PER_W, chunk_body, 0)


B_PER_W = NB // NUM_WORKERS
BCHUNK = 64
NCHUNK = B_PER_W // BCHUNK
ROWS_PER_CHUNK = BCHUNK * NFEAT


@functools.partial(
    pl.kernel,
    out_type=jax.ShapeDtypeStruct((NFEAT, DIM, NB), jnp.float32),
    mesh=_mesh,
    scratch_types=(
        pltpu.VMEM((ROWS_PER_CHUNK,), jnp.int32),
        pltpu.VMEM((ROWS_PER_CHUNK, DIM), jnp.float32),
        pltpu.VMEM((NFEAT, DIM, BCHUNK), jnp.float32),
        pltpu.SemaphoreType.DMA,
    ),
    compiler_params=pltpu.CompilerParams(
        use_tc_tiling_on_sc=False, needs_layout_passes=False),
)
def _gather_kernel(xf_hbm, table_hbm, out_hbm, idx_v, rows_v, obuf, sem):
    wid = lax.axis_index("s") * NUM_CORES + lax.axis_index("c")
    base = wid * B_PER_W * NFEAT
    lane_n = lax.iota(jnp.int32, 16) * NFEAT

    def chunk_body(c, carry):
        off = base + c * ROWS_PER_CHUNK
        pltpu.sync_copy(xf_hbm.at[pl.ds(off, ROWS_PER_CHUNK)], idx_v)
        pltpu.async_copy(table_hbm.at[idx_v], rows_v, sem).wait()

        def tr(jd, carry2):
            j = jd // DIM
            d = jd % DIM
            idx_d = jnp.full((16,), d, jnp.int32)

            def q_body(q, carry3):
                idx_r = lane_n + (q * (16 * NFEAT) + j)
                vals = plsc.load_gather(rows_v, [idx_r, idx_d])
                obuf[j, d, pl.ds(q * 16, 16)] = vals
                return carry3

            lax.fori_loop(0, BCHUNK // 16, q_body, 0)
            return carry2

        lax.fori_loop(0, NFEAT * DIM, tr, 0)
        b0 = wid * B_PER_W + c * BCHUNK
        pltpu.sync_copy(obuf, out_hbm.at[:, :, pl.ds(b0, BCHUNK)])
        return carry

    lax.fori_loop(0, NCHUNK, chunk_body, 0)


def kernel(x, table):
    tail = table[N_FULL * VCHUNK:].reshape(-1)
    table_rm = _relayout_kernel(table.T, tail).reshape(NUM_ROWS, DIM)
    out_t = _gather_kernel(x.reshape(-1), table_rm)
    return jnp.transpose(out_t, (2, 0, 1))


# speedup vs baseline: 2.4847x; 1.1350x over previous
import functools

import jax
import jax.numpy as jnp
from jax import lax
from jax.experimental import pallas as pl
from jax.experimental.pallas import tpu as pltpu
from jax.experimental.pallas import tpu_sc as plsc

NUM_ROWS = 1000000
DIM = 16
NFEAT = 26
NB = 16384

NUM_CORES = 2
NUM_SUBCORES = 16
NUM_WORKERS = NUM_CORES * NUM_SUBCORES

_mesh = plsc.VectorSubcoreMesh(
    core_axis_name="c", subcore_axis_name="s",
    num_cores=NUM_CORES, num_subcores=NUM_SUBCORES)

VCHUNK = 512
W_CHUNKS = 61
N_FULL = NUM_ROWS // VCHUNK
TAIL = NUM_ROWS - N_FULL * VCHUNK


@functools.partial(
    pl.kernel,
    out_type=jax.ShapeDtypeStruct((NUM_ROWS * DIM,), jnp.float32),
    mesh=_mesh,
    scratch_types=(
        pltpu.VMEM((DIM, VCHUNK), jnp.float32),
        pltpu.VMEM((DIM, VCHUNK), jnp.float32),
        pltpu.VMEM((VCHUNK * DIM,), jnp.float32),
        pltpu.VMEM((VCHUNK * DIM,), jnp.float32),
        pltpu.SemaphoreType.DMA,
        pltpu.SemaphoreType.DMA,
        pltpu.SemaphoreType.DMA,
        pltpu.SemaphoreType.DMA,
    ),
    compiler_params=pltpu.CompilerParams(needs_layout_passes=False),
)
def _relayout_kernel(tt_hbm, tail_hbm, out_hbm, in0, in1, r0, r1,
                     is0, is1, os0, os1):
    wid = lax.axis_index("s") * NUM_CORES + lax.axis_index("c")
    lane = lax.iota(jnp.int32, 16)
    start = wid * W_CHUNKS

    def start_in(i, buf, sem):
        pltpu.async_copy(
            tt_hbm.at[:, pl.ds((start + i) * VCHUNK, VCHUNK)], buf, sem)

    def wait_in(buf, sem):
        pltpu.make_async_copy(
            tt_hbm.at[:, pl.ds(0, VCHUNK)], buf, sem).wait()

    def start_out(i, buf, sem):
        pltpu.async_copy(
            buf, out_hbm.at[pl.ds((start + i) * (VCHUNK * DIM),
                                  VCHUNK * DIM)], sem)

    def wait_out(buf, sem):
        pltpu.make_async_copy(
            buf, out_hbm.at[pl.ds(0, VCHUNK * DIM)], sem).wait()

    def transpose(src, dst):
        def tr(v8, carry):
            base = jnp.full((16,), v8 * 8, jnp.int32)
            off = v8 * (8 * DIM)
            for u in range(8):
                row = plsc.load_gather(src, [lane, base + u])
                dst[pl.ds(off + u * DIM, DIM)] = row
            return carry

        lax.fori_loop(0, VCHUNK // 8, tr, 0, unroll=2)

    start_in(0, in0, is0)
    start_in(1, in1, is1)

    def body(k, carry):
        i0 = 2 * k

        wait_in(in0, is0)

        @pl.when(k > 0)
        def _():
            wait_out(r0, os0)

        transpose(in0, r0)
        start_out(i0, r0, os0)

        @pl.when(i0 + 2 <= W_CHUNKS - 1)
        def _():
            start_in(i0 + 2, in0, is0)

        wait_in(in1, is1)

        @pl.when(k > 0)
        def _():
            wait_out(r1, os1)

        transpose(in1, r1)
        start_out(i0 + 1, r1, os1)

        @pl.when(i0 + 3 <= W_CHUNKS - 1)
        def _():
            start_in(i0 + 3, in1, is1)

        return carry

    lax.fori_loop(0, W_CHUNKS // 2, body, 0)

    wait_in(in0, is0)
    wait_out(r0, os0)
    transpose(in0, r0)
    start_out(W_CHUNKS - 1, r0, os0)
    wait_out(r1, os1)
    wait_out(r0, os0)

    @pl.when(wid == NUM_WORKERS - 1)
    def _():
        pltpu.sync_copy(
            tt_hbm.at[:, pl.ds((N_FULL - 1) * VCHUNK, VCHUNK)], in0)
        transpose(in0, r0)
        pltpu.sync_copy(
            r0, out_hbm.at[pl.ds((N_FULL - 1) * (VCHUNK * DIM),
                                 VCHUNK * DIM)])
        pltpu.sync_copy(tail_hbm, r1.at[pl.ds(0, TAIL * DIM)])
        pltpu.sync_copy(r1.at[pl.ds(0, TAIL * DIM)],
                        out_hbm.at[pl.ds(N_FULL * VCHUNK * DIM, TAIL * DIM)])


B_PER_W = NB // NUM_WORKERS
BCHUNK = 64
NCHUNK = B_PER_W // BCHUNK
ROWS_PER_CHUNK = BCHUNK * NFEAT


@functools.partial(
    pl.kernel,
    out_type=jax.ShapeDtypeStruct((NFEAT, DIM, NB), jnp.float32),
    mesh=_mesh,
    scratch_types=(
        pltpu.VMEM((ROWS_PER_CHUNK,), jnp.int32),
        pltpu.VMEM((ROWS_PER_CHUNK, DIM), jnp.float32),
        pltpu.VMEM((NFEAT, DIM, BCHUNK), jnp.float32),
        pltpu.SemaphoreType.DMA,
    ),
    compiler_params=pltpu.CompilerParams(
        use_tc_tiling_on_sc=False, needs_layout_passes=False),
)
def _gather_kernel(xf_hbm, table_hbm, out_hbm, idx_v, rows_v, obuf, sem):
    wid = lax.axis_index("s") * NUM_CORES + lax.axis_index("c")
    base = wid * B_PER_W * NFEAT
    lane = lax.iota(jnp.int32, 16)
    lane_n = lane * NFEAT
    dsplat = [jnp.full((16,), d, jnp.int32) for d in range(DIM)]

    def chunk_body(c, carry):
        off = base + c * ROWS_PER_CHUNK
        pltpu.sync_copy(xf_hbm.at[pl.ds(off, ROWS_PER_CHUNK)], idx_v)
        pltpu.async_copy(table_hbm.at[idx_v], rows_v, sem).wait()

        def q_body(q, carry2):
            qbase = lane_n + q * (16 * NFEAT)

            def j_body(j, carry3):
                idx_r = qbase + j
                for d in range(DIM):
                    vals = plsc.load_gather(rows_v, [idx_r, dsplat[d]])
                    obuf[j, d, pl.ds(q * 16, 16)] = vals
                return carry3

            lax.fori_loop(0, NFEAT, j_body, 0, unroll=2)
            return carry2

        lax.fori_loop(0, BCHUNK // 16, q_body, 0)
        b0 = wid * B_PER_W + c * BCHUNK
        pltpu.sync_copy(obuf, out_hbm.at[:, :, pl.ds(b0, BCHUNK)])
        return carry

    lax.fori_loop(0, NCHUNK, chunk_body, 0)


def kernel(x, table):
    tail = table[N_FULL * VCHUNK:].reshape(-1)
    table_rm = _relayout_kernel(table.T, tail).reshape(NUM_ROWS, DIM)
    out_t = _gather_kernel(x.reshape(-1), table_rm)
    return jnp.transpose(out_t, (2, 0, 1))
